# baseline (speedup 1.0000x reference)
import functools

import jax
import jax.numpy as jnp
from jax import lax
from jax.experimental import pallas as pl
from jax.experimental.pallas import tpu as pltpu
from jax.experimental.pallas import tpu_sc as plsc

N = 10000
E = 320000
D = 128
B = 64
L = 3

CHUNK = 128
NCHUNKS = E // CHUNK
NW = 32
NBUF = 3
NGROUP = 27
BASE_TRIPS = NCHUNKS // NW
EXTRA = NCHUNKS % NW
FCHUNK = 400
NFCHUNKS = N // FCHUNK
FTRIPS = -(-NFCHUNKS // 16)
NZFULL = N // CHUNK
ZTRIPS = -(-(NZFULL + 1) // 16)

BLK = 1000
GRID = N // BLK


_mesh = plsc.VectorSubcoreMesh(core_axis_name="c", subcore_axis_name="s")


@functools.partial(
    pl.kernel,
    mesh=_mesh,
    out_type=jax.ShapeDtypeStruct((2, N, D), jnp.float32),
    scratch_types=[
        pltpu.VMEM((NBUF * CHUNK,), jnp.int32),
        pltpu.VMEM((NBUF * CHUNK,), jnp.int32),
        [pltpu.VMEM((CHUNK, D), jnp.float32) for _ in range(NBUF)],
        pltpu.VMEM_SHARED((N, D), jnp.float32),
        [pltpu.SemaphoreType.DMA for _ in range(NBUF)],
        [pltpu.SemaphoreType.DMA for _ in range(NBUF)],
    ],
)
def _sc_aggregate(h_hbm, src_hbm, dst_hbm, out_hbm, sidx, didx, rows,
                  acc, gsem, ssem):
    c = lax.axis_index("c")
    s = lax.axis_index("s")
    wid = s * 2 + c

    zeros16 = jnp.zeros((16,), jnp.float32)

    def zrow(r, carry):
        for cb in range(D // 16):
            rows[0][r, pl.ds(cb * 16, 16)] = zeros16
        return carry

    lax.fori_loop(0, CHUNK, zrow, 0)

    def zcopy(k, carry):
        cid = s + k * 16

        @pl.when(cid < NZFULL)
        def _():
            pltpu.sync_copy(rows[0], acc.at[pl.ds(cid * CHUNK, CHUNK)])

        @pl.when(cid == NZFULL)
        def _():
            pltpu.sync_copy(rows[0].at[pl.ds(0, N - NZFULL * CHUNK)],
                            acc.at[pl.ds(NZFULL * CHUNK, N - NZFULL * CHUNK)])

        return carry

    lax.fori_loop(0, ZTRIPS, zcopy, 0)

    plsc.subcore_barrier()

    cid0 = BASE_TRIPS * wid + jnp.minimum(wid, EXTRA)
    ntrips = jnp.where(wid < EXTRA, BASE_TRIPS + 1, BASE_TRIPS)

    def body(g, carry):
        for b in range(NBUF):
            @pl.when((g > 0) & (g * NBUF + b - NBUF < ntrips))
            def _(b=b):
                dj = didx.at[pl.ds(b * CHUNK, CHUNK)]
                pltpu.make_async_copy(rows[b], acc.at[dj], ssem[b]).wait()

        base_g = (cid0 + g * NBUF) * CHUNK
        pltpu.sync_copy(src_hbm.at[pl.ds(base_g, NBUF * CHUNK)], sidx)
        pltpu.sync_copy(dst_hbm.at[pl.ds(base_g, NBUF * CHUNK)], didx)
        for b in range(NBUF):
            t = g * NBUF + b

            @pl.when(t < ntrips)
            def _(b=b, t=t):
                idx_b = sidx.at[pl.ds(b * CHUNK, CHUNK)]
                pltpu.async_copy(h_hbm.at[idx_b], rows[b], gsem[b])

        for b in range(NBUF):
            t = g * NBUF + b

            @pl.when(t < ntrips)
            def _(b=b, t=t):
                idx_b = sidx.at[pl.ds(b * CHUNK, CHUNK)]
                dj = didx.at[pl.ds(b * CHUNK, CHUNK)]
                pltpu.make_async_copy(h_hbm.at[idx_b], rows[b],
                                      gsem[b]).wait()
                pltpu.async_copy(rows[b], acc.at[dj], ssem[b], add=True)

        return carry

    lax.fori_loop(0, NGROUP, body, 0)

    for b in range(NBUF):
        @pl.when((NGROUP - 1) * NBUF + b < ntrips)
        def _(b=b):
            dj = didx.at[pl.ds(b * CHUNK, CHUNK)]
            pltpu.make_async_copy(rows[b], acc.at[dj], ssem[b]).wait()

    plsc.subcore_barrier()

    def wcopy(k, carry):
        cid = s + k * 16

        @pl.when(cid < NFCHUNKS)
        def _():
            r = cid * FCHUNK
            pltpu.sync_copy(acc.at[pl.ds(r, FCHUNK)],
                            out_hbm.at[c, pl.ds(r, FCHUNK)])

        return carry

    lax.fori_loop(0, FTRIPS, wcopy, 0)


def _mlp_stats_body(h_ref, a0_ref, a1_ref, sc_ref, w1_ref, b1_ref, w2_ref,
                    b2_ref, y_ref, st_ref):
    i = pl.program_id(0)
    u = h_ref[...] * sc_ref[...] + a0_ref[...] + a1_ref[...]
    t = lax.dot_general(u, w1_ref[...], (((1,), (0,)), ((), ())),
                        preferred_element_type=jnp.float32) + b1_ref[...]
    t = jnp.maximum(t, 0.0)
    y = lax.dot_general(t, w2_ref[...], (((1,), (0,)), ((), ())),
                        preferred_element_type=jnp.float32) + b2_ref[...]
    y_ref[...] = y
    ps = jnp.concatenate(
        [jnp.sum(y, 0, keepdims=True), jnp.sum(y * y, 0, keepdims=True)], 0)

    @pl.when(i == 0)
    def _():
        st_ref[...] = ps

    @pl.when(i != 0)
    def _():
        st_ref[...] = st_ref[...] + ps


def _bn_body(y_ref, st_ref, g_ref, be_ref, o_ref):
    mean = st_ref[0:1, :] * (1.0 / N)
    var = st_ref[1:2, :] * (1.0 / N) - mean * mean
    inv = lax.rsqrt(var + 1e-5)
    o_ref[...] = jnp.maximum(
        (y_ref[...] - mean) * inv * g_ref[...] + be_ref[...], 0.0)


def _bn_pool_body(y_ref, st_ref, g_ref, be_ref, b_ref, o_ref, sums, cnt):
    i = pl.program_id(0)
    mean = st_ref[0:1, :] * (1.0 / N)
    var = st_ref[1:2, :] * (1.0 / N) - mean * mean
    inv = lax.rsqrt(var + 1e-5)
    hn = jnp.maximum(
        (y_ref[...] - mean) * inv * g_ref[...] + be_ref[...], 0.0)
    bi = b_ref[...][0]
    oh = (bi == lax.broadcasted_iota(jnp.int32, (B, BLK), 0))
    oh = oh.astype(jnp.float32)
    psum = lax.dot_general(oh, hn, (((1,), (0,)), ((), ())),
                           preferred_element_type=jnp.float32)
    pcnt = jnp.broadcast_to(jnp.sum(oh, axis=1, keepdims=True), (B, D))

    @pl.when(i == 0)
    def _():
        sums[...] = psum
        cnt[...] = pcnt

    @pl.when(i != 0)
    def _():
        sums[...] = sums[...] + psum
        cnt[...] = cnt[...] + pcnt

    o_ref[...] = sums[...] / jnp.maximum(cnt[...], 1.0)


_row_spec = pl.BlockSpec((BLK, D), lambda i: (i, 0))
_const = lambda shape: pl.BlockSpec(shape, lambda i: (0,) * len(shape))

_mlp_stats = pl.pallas_call(
    _mlp_stats_body,
    grid=(GRID,),
    in_specs=[_row_spec, _row_spec, _row_spec, _const((1, D)),
              _const((D, D)), _const((1, D)), _const((D, D)), _const((1, D))],
    out_specs=[_row_spec, _const((2, D))],
    out_shape=[jax.ShapeDtypeStruct((N, D), jnp.float32),
               jax.ShapeDtypeStruct((2, D), jnp.float32)],
)

_bn = pl.pallas_call(
    _bn_body,
    grid=(GRID,),
    in_specs=[_row_spec, _const((2, D)), _const((1, D)), _const((1, D))],
    out_specs=_row_spec,
    out_shape=jax.ShapeDtypeStruct((N, D), jnp.float32),
)

_bn_pool = pl.pallas_call(
    _bn_pool_body,
    grid=(GRID,),
    in_specs=[_row_spec, _const((2, D)), _const((1, D)), _const((1, D)),
              pl.BlockSpec((1, 1, BLK), lambda i: (i, 0, 0))],
    out_specs=_const((B, D)),
    out_shape=jax.ShapeDtypeStruct((B, D), jnp.float32),
    scratch_shapes=[pltpu.VMEM((B, D), jnp.float32),
                    pltpu.VMEM((B, D), jnp.float32)],
)


def kernel(x, edge_index, batch, eps, W1, b1, W2, b2, gamma, beta):
    pad = jnp.zeros((NBUF * CHUNK,), jnp.int32)
    src = jnp.concatenate([edge_index[0], pad])
    dst = jnp.concatenate([edge_index[1], pad])
    batch3 = batch.reshape(GRID, 1, BLK)
    ones_row = jnp.ones((1, D), jnp.float32)

    h = x
    out = None
    for i in range(L):
        parts = _sc_aggregate(h, src, dst)
        scale_row = (1.0 + eps[i]) * ones_row
        y, st = _mlp_stats(h, parts[0], parts[1], scale_row, W1[i],
                           b1[i].reshape(1, D), W2[i], b2[i].reshape(1, D))
        g = gamma[i].reshape(1, D)
        be = beta[i].reshape(1, D)
        if i < L - 1:
            h = _bn(y, st, g, be)
        else:
            out = _bn_pool(y, st, g, be, batch3)
    return out


# speedup vs baseline: 1.1576x; 1.1576x over previous
import functools

import jax
import jax.numpy as jnp
from jax import lax
from jax.experimental import pallas as pl
from jax.experimental.pallas import tpu as pltpu
from jax.experimental.pallas import tpu_sc as plsc

N = 10000
E = 320000
D = 128
B = 64
L = 3

CHUNK = 128
NCHUNKS = E // CHUNK
NW = 32
NBUF = 3
NGROUP = 28
BASE_TRIPS = NCHUNKS // NW
EXTRA = NCHUNKS % NW
FCHUNK = 400
NFCHUNKS = N // FCHUNK
FTRIPS = -(-NFCHUNKS // 16)
NZFULL = N // CHUNK
ZTRIPS = -(-(NZFULL + 1) // 16)

BLK = 1000
GRID = N // BLK


_mesh = plsc.VectorSubcoreMesh(core_axis_name="c", subcore_axis_name="s")


@functools.partial(
    pl.kernel,
    mesh=_mesh,
    out_type=jax.ShapeDtypeStruct((2, N, D), jnp.float32),
    scratch_types=[
        [pltpu.VMEM((NBUF * CHUNK,), jnp.int32) for _ in range(2)],
        [pltpu.VMEM((NBUF * CHUNK,), jnp.int32) for _ in range(2)],
        [pltpu.VMEM((CHUNK, D), jnp.float32) for _ in range(NBUF)],
        pltpu.VMEM_SHARED((N, D), jnp.float32),
        [pltpu.SemaphoreType.DMA for _ in range(NBUF)],
        [pltpu.SemaphoreType.DMA for _ in range(NBUF)],
    ],
)
def _sc_aggregate(h_hbm, src_hbm, dst_hbm, out_hbm, sidx, didx, rows,
                  acc, gsem, ssem):
    c = lax.axis_index("c")
    s = lax.axis_index("s")
    wid = s * 2 + c

    zeros16 = jnp.zeros((16,), jnp.float32)

    def zrow(r, carry):
        for cb in range(D // 16):
            rows[0][r, pl.ds(cb * 16, 16)] = zeros16
        return carry

    lax.fori_loop(0, CHUNK, zrow, 0)

    def zcopy(k, carry):
        cid = s + k * 16

        @pl.when(cid < NZFULL)
        def _():
            pltpu.sync_copy(rows[0], acc.at[pl.ds(cid * CHUNK, CHUNK)])

        @pl.when(cid == NZFULL)
        def _():
            pltpu.sync_copy(rows[0].at[pl.ds(0, N - NZFULL * CHUNK)],
                            acc.at[pl.ds(NZFULL * CHUNK, N - NZFULL * CHUNK)])

        return carry

    lax.fori_loop(0, ZTRIPS, zcopy, 0)

    plsc.subcore_barrier()

    cid0 = BASE_TRIPS * wid + jnp.minimum(wid, EXTRA)
    ntrips = jnp.where(wid < EXTRA, BASE_TRIPS + 1, BASE_TRIPS)

    pltpu.sync_copy(src_hbm.at[pl.ds(cid0 * CHUNK, NBUF * CHUNK)], sidx[0])
    pltpu.sync_copy(dst_hbm.at[pl.ds(cid0 * CHUNK, NBUF * CHUNK)], didx[0])

    def group(g, p):
        for b in range(NBUF):
            @pl.when((g >= 1) & ((g - 1) * NBUF + b < ntrips))
            def _(b=b):
                dj = didx[1 - p].at[pl.ds(b * CHUNK, CHUNK)]
                pltpu.make_async_copy(rows[b], acc.at[dj], ssem[b]).wait()

        for b in range(NBUF):
            @pl.when(g * NBUF + b < ntrips)
            def _(b=b):
                idx_b = sidx[p].at[pl.ds(b * CHUNK, CHUNK)]
                pltpu.async_copy(h_hbm.at[idx_b], rows[b], gsem[b])

        @pl.when((g + 1) * NBUF < ntrips)
        def _():
            base_n = (cid0 + (g + 1) * NBUF) * CHUNK
            pltpu.sync_copy(src_hbm.at[pl.ds(base_n, NBUF * CHUNK)],
                            sidx[1 - p])
            pltpu.sync_copy(dst_hbm.at[pl.ds(base_n, NBUF * CHUNK)],
                            didx[1 - p])

        for b in range(NBUF):
            @pl.when(g * NBUF + b < ntrips)
            def _(b=b):
                idx_b = sidx[p].at[pl.ds(b * CHUNK, CHUNK)]
                dj = didx[p].at[pl.ds(b * CHUNK, CHUNK)]
                pltpu.make_async_copy(h_hbm.at[idx_b], rows[b],
                                      gsem[b]).wait()
                pltpu.async_copy(rows[b], acc.at[dj], ssem[b], add=True)

    def pair(k, carry):
        group(2 * k, 0)
        group(2 * k + 1, 1)
        return carry

    lax.fori_loop(0, NGROUP // 2, pair, 0)

    plsc.subcore_barrier()

    def wcopy(k, carry):
        cid = s + k * 16

        @pl.when(cid < NFCHUNKS)
        def _():
            r = cid * FCHUNK
            pltpu.sync_copy(acc.at[pl.ds(r, FCHUNK)],
                            out_hbm.at[c, pl.ds(r, FCHUNK)])

        return carry

    lax.fori_loop(0, FTRIPS, wcopy, 0)


def _mlp_stats_body(h_ref, a0_ref, a1_ref, sc_ref, w1_ref, b1_ref, w2_ref,
                    b2_ref, y_ref, st_ref):
    i = pl.program_id(0)
    u = h_ref[...] * sc_ref[...] + a0_ref[...] + a1_ref[...]
    t = lax.dot_general(u, w1_ref[...], (((1,), (0,)), ((), ())),
                        preferred_element_type=jnp.float32) + b1_ref[...]
    t = jnp.maximum(t, 0.0)
    y = lax.dot_general(t, w2_ref[...], (((1,), (0,)), ((), ())),
                        preferred_element_type=jnp.float32) + b2_ref[...]
    y_ref[...] = y
    ps = jnp.concatenate(
        [jnp.sum(y, 0, keepdims=True), jnp.sum(y * y, 0, keepdims=True)], 0)

    @pl.when(i == 0)
    def _():
        st_ref[...] = ps

    @pl.when(i != 0)
    def _():
        st_ref[...] = st_ref[...] + ps


def _bn_body(y_ref, st_ref, g_ref, be_ref, o_ref):
    mean = st_ref[0:1, :] * (1.0 / N)
    var = st_ref[1:2, :] * (1.0 / N) - mean * mean
    inv = lax.rsqrt(var + 1e-5)
    o_ref[...] = jnp.maximum(
        (y_ref[...] - mean) * inv * g_ref[...] + be_ref[...], 0.0)


def _bn_pool_body(y_ref, st_ref, g_ref, be_ref, b_ref, o_ref, sums, cnt):
    i = pl.program_id(0)
    mean = st_ref[0:1, :] * (1.0 / N)
    var = st_ref[1:2, :] * (1.0 / N) - mean * mean
    inv = lax.rsqrt(var + 1e-5)
    hn = jnp.maximum(
        (y_ref[...] - mean) * inv * g_ref[...] + be_ref[...], 0.0)
    bi = b_ref[...][0]
    oh = (bi == lax.broadcasted_iota(jnp.int32, (B, BLK), 0))
    oh = oh.astype(jnp.float32)
    psum = lax.dot_general(oh, hn, (((1,), (0,)), ((), ())),
                           preferred_element_type=jnp.float32)
    pcnt = jnp.broadcast_to(jnp.sum(oh, axis=1, keepdims=True), (B, D))

    @pl.when(i == 0)
    def _():
        sums[...] = psum
        cnt[...] = pcnt

    @pl.when(i != 0)
    def _():
        sums[...] = sums[...] + psum
        cnt[...] = cnt[...] + pcnt

    o_ref[...] = sums[...] / jnp.maximum(cnt[...], 1.0)


_row_spec = pl.BlockSpec((BLK, D), lambda i: (i, 0))
_const = lambda shape: pl.BlockSpec(shape, lambda i: (0,) * len(shape))

_mlp_stats = pl.pallas_call(
    _mlp_stats_body,
    grid=(GRID,),
    in_specs=[_row_spec, _row_spec, _row_spec, _const((1, D)),
              _const((D, D)), _const((1, D)), _const((D, D)), _const((1, D))],
    out_specs=[_row_spec, _const((2, D))],
    out_shape=[jax.ShapeDtypeStruct((N, D), jnp.float32),
               jax.ShapeDtypeStruct((2, D), jnp.float32)],
)

_bn = pl.pallas_call(
    _bn_body,
    grid=(GRID,),
    in_specs=[_row_spec, _const((2, D)), _const((1, D)), _const((1, D))],
    out_specs=_row_spec,
    out_shape=jax.ShapeDtypeStruct((N, D), jnp.float32),
)

_bn_pool = pl.pallas_call(
    _bn_pool_body,
    grid=(GRID,),
    in_specs=[_row_spec, _const((2, D)), _const((1, D)), _const((1, D)),
              pl.BlockSpec((1, 1, BLK), lambda i: (i, 0, 0))],
    out_specs=_const((B, D)),
    out_shape=jax.ShapeDtypeStruct((B, D), jnp.float32),
    scratch_shapes=[pltpu.VMEM((B, D), jnp.float32),
                    pltpu.VMEM((B, D), jnp.float32)],
)


def kernel(x, edge_index, batch, eps, W1, b1, W2, b2, gamma, beta):
    pad = jnp.zeros((NBUF * CHUNK,), jnp.int32)
    src = jnp.concatenate([edge_index[0], pad])
    dst = jnp.concatenate([edge_index[1], pad])
    batch3 = batch.reshape(GRID, 1, BLK)
    ones_row = jnp.ones((1, D), jnp.float32)

    h = x
    out = None
    for i in range(L):
        parts = _sc_aggregate(h, src, dst)
        scale_row = (1.0 + eps[i]) * ones_row
        y, st = _mlp_stats(h, parts[0], parts[1], scale_row, W1[i],
                           b1[i].reshape(1, D), W2[i], b2[i].reshape(1, D))
        g = gamma[i].reshape(1, D)
        be = beta[i].reshape(1, D)
        if i < L - 1:
            h = _bn(y, st, g, be)
        else:
            out = _bn_pool(y, st, g, be, batch3)
    return out


# speedup vs baseline: 1.2352x; 1.0670x over previous
import functools

import jax
import jax.numpy as jnp
from jax import lax
from jax.experimental import pallas as pl
from jax.experimental.pallas import tpu as pltpu
from jax.experimental.pallas import tpu_sc as plsc

N = 10000
E = 320000
D = 128
B = 64
L = 3

CHUNK = 128
NCHUNKS = E // CHUNK
NW = 32
NBUF = 3
NGROUP = 28
BASE_TRIPS = NCHUNKS // NW
EXTRA = NCHUNKS % NW
FCHUNK = 400
NFCHUNKS = N // FCHUNK
FTRIPS = -(-NFCHUNKS // 16)
NZFULL = N // CHUNK
ZTRIPS = -(-(NZFULL + 1) // 16)

BLK = 1000
GRID = N // BLK


_mesh = plsc.VectorSubcoreMesh(core_axis_name="c", subcore_axis_name="s")


@functools.partial(
    pl.kernel,
    mesh=_mesh,
    out_type=jax.ShapeDtypeStruct((2, N, D), jnp.float32),
    scratch_types=[
        [pltpu.VMEM((NBUF * CHUNK,), jnp.int32) for _ in range(2)],
        [pltpu.VMEM((NBUF * CHUNK,), jnp.int32) for _ in range(2)],
        [pltpu.VMEM((CHUNK, D), jnp.float32) for _ in range(NBUF)],
        pltpu.VMEM_SHARED((N, D), jnp.float32),
        [pltpu.SemaphoreType.DMA for _ in range(NBUF)],
        [pltpu.SemaphoreType.DMA for _ in range(NBUF)],
    ],
)
def _sc_aggregate(h_hbm, src_hbm, dst_hbm, out_hbm, sidx, didx, rows,
                  acc, gsem, ssem):
    c = lax.axis_index("c")
    s = lax.axis_index("s")
    wid = s * 2 + c

    zeros16 = jnp.zeros((16,), jnp.float32)

    def zrow(r, carry):
        for cb in range(D // 16):
            rows[0][r, pl.ds(cb * 16, 16)] = zeros16
        return carry

    lax.fori_loop(0, CHUNK, zrow, 0)

    def zcopy(k, carry):
        cid = s + k * 16

        @pl.when(cid < NZFULL)
        def _():
            pltpu.sync_copy(rows[0], acc.at[pl.ds(cid * CHUNK, CHUNK)])

        @pl.when(cid == NZFULL)
        def _():
            pltpu.sync_copy(rows[0].at[pl.ds(0, N - NZFULL * CHUNK)],
                            acc.at[pl.ds(NZFULL * CHUNK, N - NZFULL * CHUNK)])

        return carry

    lax.fori_loop(0, ZTRIPS, zcopy, 0)

    plsc.subcore_barrier()

    cid0 = BASE_TRIPS * wid + jnp.minimum(wid, EXTRA)
    ntrips = jnp.where(wid < EXTRA, BASE_TRIPS + 1, BASE_TRIPS)

    pltpu.sync_copy(src_hbm.at[pl.ds(cid0 * CHUNK, NBUF * CHUNK)], sidx[0])
    pltpu.sync_copy(dst_hbm.at[pl.ds(cid0 * CHUNK, NBUF * CHUNK)], didx[0])

    def group(g, p):
        for b in range(NBUF):
            @pl.when((g >= 1) & ((g - 1) * NBUF + b < ntrips))
            def _(b=b):
                dj = didx[1 - p].at[pl.ds(b * CHUNK, CHUNK)]
                pltpu.make_async_copy(rows[b], acc.at[dj], ssem[b]).wait()

            @pl.when(g * NBUF + b < ntrips)
            def _(b=b):
                idx_b = sidx[p].at[pl.ds(b * CHUNK, CHUNK)]
                pltpu.async_copy(h_hbm.at[idx_b], rows[b], gsem[b])

        @pl.when((g + 1) * NBUF < ntrips)
        def _():
            base_n = (cid0 + (g + 1) * NBUF) * CHUNK
            pltpu.sync_copy(src_hbm.at[pl.ds(base_n, NBUF * CHUNK)],
                            sidx[1 - p])
            pltpu.sync_copy(dst_hbm.at[pl.ds(base_n, NBUF * CHUNK)],
                            didx[1 - p])

        for b in range(NBUF):
            @pl.when(g * NBUF + b < ntrips)
            def _(b=b):
                idx_b = sidx[p].at[pl.ds(b * CHUNK, CHUNK)]
                dj = didx[p].at[pl.ds(b * CHUNK, CHUNK)]
                pltpu.make_async_copy(h_hbm.at[idx_b], rows[b],
                                      gsem[b]).wait()
                pltpu.async_copy(rows[b], acc.at[dj], ssem[b], add=True)

    def pair(k, carry):
        group(2 * k, 0)
        group(2 * k + 1, 1)
        return carry

    lax.fori_loop(0, NGROUP // 2, pair, 0)

    plsc.subcore_barrier()

    def wcopy(k, carry):
        cid = s + k * 16

        @pl.when(cid < NFCHUNKS)
        def _():
            r = cid * FCHUNK
            pltpu.sync_copy(acc.at[pl.ds(r, FCHUNK)],
                            out_hbm.at[c, pl.ds(r, FCHUNK)])

        return carry

    lax.fori_loop(0, FTRIPS, wcopy, 0)


def _mlp_stats_body(h_ref, a0_ref, a1_ref, sc_ref, w1_ref, b1_ref, w2_ref,
                    b2_ref, y_ref, st_ref):
    i = pl.program_id(0)
    u = h_ref[...] * sc_ref[...] + a0_ref[...] + a1_ref[...]
    t = lax.dot_general(u, w1_ref[...], (((1,), (0,)), ((), ())),
                        preferred_element_type=jnp.float32) + b1_ref[...]
    t = jnp.maximum(t, 0.0)
    y = lax.dot_general(t, w2_ref[...], (((1,), (0,)), ((), ())),
                        preferred_element_type=jnp.float32) + b2_ref[...]
    y_ref[...] = y
    ps = jnp.concatenate(
        [jnp.sum(y, 0, keepdims=True), jnp.sum(y * y, 0, keepdims=True)], 0)

    @pl.when(i == 0)
    def _():
        st_ref[...] = ps

    @pl.when(i != 0)
    def _():
        st_ref[...] = st_ref[...] + ps


def _bn_body(y_ref, st_ref, g_ref, be_ref, o_ref):
    mean = st_ref[0:1, :] * (1.0 / N)
    var = st_ref[1:2, :] * (1.0 / N) - mean * mean
    inv = lax.rsqrt(var + 1e-5)
    o_ref[...] = jnp.maximum(
        (y_ref[...] - mean) * inv * g_ref[...] + be_ref[...], 0.0)


def _bn_pool_body(y_ref, st_ref, g_ref, be_ref, b_ref, o_ref, sums, cnt):
    i = pl.program_id(0)
    mean = st_ref[0:1, :] * (1.0 / N)
    var = st_ref[1:2, :] * (1.0 / N) - mean * mean
    inv = lax.rsqrt(var + 1e-5)
    hn = jnp.maximum(
        (y_ref[...] - mean) * inv * g_ref[...] + be_ref[...], 0.0)
    bi = b_ref[...][0]
    oh = (bi == lax.broadcasted_iota(jnp.int32, (B, BLK), 0))
    oh = oh.astype(jnp.float32)
    psum = lax.dot_general(oh, hn, (((1,), (0,)), ((), ())),
                           preferred_element_type=jnp.float32)
    pcnt = jnp.broadcast_to(jnp.sum(oh, axis=1, keepdims=True), (B, D))

    @pl.when(i == 0)
    def _():
        sums[...] = psum
        cnt[...] = pcnt

    @pl.when(i != 0)
    def _():
        sums[...] = sums[...] + psum
        cnt[...] = cnt[...] + pcnt

    o_ref[...] = sums[...] / jnp.maximum(cnt[...], 1.0)


_row_spec = pl.BlockSpec((BLK, D), lambda i: (i, 0))
_const = lambda shape: pl.BlockSpec(shape, lambda i: (0,) * len(shape))

_mlp_stats = pl.pallas_call(
    _mlp_stats_body,
    grid=(GRID,),
    in_specs=[_row_spec, _row_spec, _row_spec, _const((1, D)),
              _const((D, D)), _const((1, D)), _const((D, D)), _const((1, D))],
    out_specs=[_row_spec, _const((2, D))],
    out_shape=[jax.ShapeDtypeStruct((N, D), jnp.float32),
               jax.ShapeDtypeStruct((2, D), jnp.float32)],
)

_bn = pl.pallas_call(
    _bn_body,
    grid=(GRID,),
    in_specs=[_row_spec, _const((2, D)), _const((1, D)), _const((1, D))],
    out_specs=_row_spec,
    out_shape=jax.ShapeDtypeStruct((N, D), jnp.float32),
)

_bn_pool = pl.pallas_call(
    _bn_pool_body,
    grid=(GRID,),
    in_specs=[_row_spec, _const((2, D)), _const((1, D)), _const((1, D)),
              pl.BlockSpec((1, 1, BLK), lambda i: (i, 0, 0))],
    out_specs=_const((B, D)),
    out_shape=jax.ShapeDtypeStruct((B, D), jnp.float32),
    scratch_shapes=[pltpu.VMEM((B, D), jnp.float32),
                    pltpu.VMEM((B, D), jnp.float32)],
)


def kernel(x, edge_index, batch, eps, W1, b1, W2, b2, gamma, beta):
    pad = jnp.zeros((NBUF * CHUNK,), jnp.int32)
    src = jnp.concatenate([edge_index[0], pad])
    dst = jnp.concatenate([edge_index[1], pad])
    batch3 = batch.reshape(GRID, 1, BLK)
    ones_row = jnp.ones((1, D), jnp.float32)

    h = x
    out = None
    for i in range(L):
        parts = _sc_aggregate(h, src, dst)
        scale_row = (1.0 + eps[i]) * ones_row
        y, st = _mlp_stats(h, parts[0], parts[1], scale_row, W1[i],
                           b1[i].reshape(1, D), W2[i], b2[i].reshape(1, D))
        g = gamma[i].reshape(1, D)
        be = beta[i].reshape(1, D)
        if i < L - 1:
            h = _bn(y, st, g, be)
        else:
            out = _bn_pool(y, st, g, be, batch3)
    return out
